# Initial kernel scaffold; baseline (speedup 1.0000x reference)
import jax
import jax.numpy as jnp
from jax.experimental import pallas as pl


def kernel(x, edge_index, Wl0, bl0, Wr0, Wl1, bl1, Wr1, Wc, bc, Wa1, ba1, Wa2, ba2):
    raise NotImplementedError("write your pallas kernel here")


# speedup vs baseline: 3.0403x; 3.0403x over previous
import jax
import jax.numpy as jnp
from jax import lax
from jax.experimental import pallas as pl
from jax.experimental.pallas import tpu as pltpu
from jax.experimental.pallas import tpu_sc as plsc

N = 10000
E = 320000
D = 128
NC, NS = 2, 16
NW = NC * NS
C = 128
K = 80
E_PAD = NW * K * C
NACC = 10112
RPT = NACC // NS
BLK = 1000
GRID = N // BLK


def _sc_mesh():
    return plsc.VectorSubcoreMesh(
        core_axis_name="c", subcore_axis_name="s",
        num_cores=NC, num_subcores=NS)


def _decode_chunk(combo_v, j, src_c, dst_c, b):
    for k in range(C // 16):
        cv = combo_v[j, pl.ds(k * 16, 16)]
        if src_c is not None:
            src_c[b, pl.ds(k * 16, 16)] = lax.shift_right_logical(
                cv, jnp.int32(14))
        dst_c[b, pl.ds(k * 16, 16)] = lax.bitwise_and(cv, jnp.int32(16383))


def _make_sc_aggregate():
    scratch = [
        pltpu.VMEM((K, C), jnp.int32),
        pltpu.VMEM((2, C), jnp.int32),
        pltpu.VMEM((2, C), jnp.int32),
        pltpu.VMEM((2, C, D), jnp.float32),
        pltpu.VMEM_SHARED((NACC, D), jnp.float32),
        pltpu.SemaphoreType.DMA,
        pltpu.SemaphoreType.DMA,
    ]

    def body(table, combo2d, zrow, agg_out,
             combo_v, src_c, dst_c, rows, acc, sem0, sem1):
        sems = (sem0, sem1)
        cid = lax.axis_index("c")
        sid = lax.axis_index("s")
        wid = sid * NC + cid
        r0 = sid * RPT

        pltpu.sync_copy(zrow.at[pl.ds(r0, RPT)], acc.at[pl.ds(r0, RPT)])
        pltpu.sync_copy(combo2d.at[pl.ds(wid * K, K)], combo_v)
        plsc.subcore_barrier()

        def launch(j, b):
            _decode_chunk(combo_v, j, src_c, dst_c, b)
            pltpu.async_copy(table.at[src_c.at[b]], rows.at[b], sems[b])

        def drain(b):
            pltpu.make_async_copy(
                table.at[src_c.at[b]], rows.at[b], sems[b]).wait()
            pltpu.sync_copy(rows.at[b], acc.at[dst_c.at[b]], add=True)

        launch(0, 0)

        def step(i, carry):
            j = i * 2
            launch(j + 1, 1)
            drain(0)

            @pl.when(j + 2 < K)
            def _():
                launch(j + 2, 0)

            drain(1)
            return carry

        lax.fori_loop(0, K // 2, step, 0)
        plsc.subcore_barrier()

        pltpu.sync_copy(acc.at[pl.ds(r0, RPT)],
                        agg_out.at[cid, pl.ds(r0, RPT)])

    return pl.kernel(
        body,
        out_type=jax.ShapeDtypeStruct((NC, NACC, D), jnp.float32),
        mesh=_sc_mesh(),
        scratch_types=scratch,
    )


def _make_sc_degree():
    scratch = [
        pltpu.VMEM((K, C), jnp.int32),
        pltpu.VMEM((1, C), jnp.int32),
        pltpu.VMEM((C, D), jnp.float32),
        pltpu.VMEM_SHARED((NACC, D), jnp.float32),
    ]

    def body(combo2d, zrow, o128, deg_out, combo_v, dst_c, ones_v, degt):
        cid = lax.axis_index("c")
        sid = lax.axis_index("s")
        wid = sid * NC + cid
        r0 = sid * RPT

        pltpu.sync_copy(zrow.at[pl.ds(r0, RPT)], degt.at[pl.ds(r0, RPT)])
        pltpu.sync_copy(combo2d.at[pl.ds(wid * K, K)], combo_v)
        pltpu.sync_copy(o128, ones_v)
        plsc.subcore_barrier()

        def step(j, carry):
            _decode_chunk(combo_v, j, None, dst_c, 0)
            pltpu.sync_copy(ones_v, degt.at[dst_c.at[0]], add=True)
            return carry

        lax.fori_loop(0, K, step, 0)
        plsc.subcore_barrier()
        pltpu.sync_copy(degt.at[pl.ds(r0, RPT)],
                        deg_out.at[cid, pl.ds(r0, RPT)])

    return pl.kernel(
        body,
        out_type=jax.ShapeDtypeStruct((NC, NACC, D), jnp.float32),
        mesh=_sc_mesh(),
        scratch_types=scratch,
    )


def _deg_from(deg_ref):
    return jnp.sum(deg_ref[0] + deg_ref[1], axis=1, keepdims=True) * (1.0 / D)


def _dot(a, b):
    return jnp.dot(a, b, preferred_element_type=jnp.float32,
                   precision=lax.Precision.HIGHEST)


def _sage_dense(agg2, deg8, x, Wl, bl, Wr):
    def body(agg_ref, deg_ref, x_ref, wl_ref, bl_ref, wr_ref, out_ref):
        acc = agg_ref[0] + agg_ref[1]
        deg = jnp.maximum(_deg_from(deg_ref), 1.0)
        mean = acc / deg
        h = _dot(mean, wl_ref[...]) + bl_ref[...] + _dot(x_ref[...], wr_ref[...])
        out_ref[...] = jnp.maximum(h, 0.0)

    return pl.pallas_call(
        body,
        grid=(GRID,),
        in_specs=[
            pl.BlockSpec((NC, BLK, D), lambda i: (0, i, 0)),
            pl.BlockSpec((NC, BLK, D), lambda i: (0, i, 0)),
            pl.BlockSpec((BLK, D), lambda i: (i, 0)),
            pl.BlockSpec((D, D), lambda i: (0, 0)),
            pl.BlockSpec((1, D), lambda i: (0, 0)),
            pl.BlockSpec((D, D), lambda i: (0, 0)),
        ],
        out_specs=pl.BlockSpec((BLK, D), lambda i: (i, 0)),
        out_shape=jax.ShapeDtypeStruct((N, D), jnp.float32),
    )(agg2, deg8, x, Wl, bl, Wr)


def _heads(agg2, deg8, h, Wl, bl, Wr, Wcp, bcp, Wa1p, ba1p, Wa2p, ba2p):
    def body(agg_ref, deg_ref, h_ref, wl_ref, bl_ref, wr_ref,
             wc_ref, bc_ref, wa1_ref, ba1_ref, wa2_ref, ba2_ref,
             pred_ref, adv_ref):
        acc = agg_ref[0] + agg_ref[1]
        deg = jnp.maximum(_deg_from(deg_ref), 1.0)
        mean = acc / deg
        h2 = _dot(mean, wl_ref[...]) + bl_ref[...] + _dot(h_ref[...], wr_ref[...])
        h2 = jnp.maximum(h2, 0.0)
        pred_ref[...] = _dot(h2, wc_ref[...]) + bc_ref[...]
        z = jnp.maximum(_dot(h2, wa1_ref[...]) + ba1_ref[...], 0.0)
        adv_ref[...] = _dot(z, wa2_ref[...]) + ba2_ref[...]

    full = lambda i: (0, 0)
    return pl.pallas_call(
        body,
        grid=(GRID,),
        in_specs=[
            pl.BlockSpec((NC, BLK, D), lambda i: (0, i, 0)),
            pl.BlockSpec((NC, BLK, D), lambda i: (0, i, 0)),
            pl.BlockSpec((BLK, D), lambda i: (i, 0)),
            pl.BlockSpec((D, D), full),
            pl.BlockSpec((1, D), full),
            pl.BlockSpec((D, D), full),
            pl.BlockSpec((D, D), full),
            pl.BlockSpec((1, D), full),
            pl.BlockSpec((D, D), full),
            pl.BlockSpec((1, D), full),
            pl.BlockSpec((D, D), full),
            pl.BlockSpec((1, D), full),
        ],
        out_specs=[
            pl.BlockSpec((BLK, D), lambda i: (i, 0)),
            pl.BlockSpec((BLK, D), lambda i: (i, 0)),
        ],
        out_shape=[
            jax.ShapeDtypeStruct((N, D), jnp.float32),
            jax.ShapeDtypeStruct((N, D), jnp.float32),
        ],
    )(agg2, deg8, h, Wl, bl, Wr, Wcp, bcp, Wa1p, ba1p, Wa2p, ba2p)


_agg = _make_sc_aggregate()
_deg = _make_sc_degree()


def kernel(x, edge_index, Wl0, bl0, Wr0, Wl1, bl1, Wr1, Wc, bc, Wa1, ba1,
           Wa2, ba2):
    src = edge_index[0].astype(jnp.int32)
    dst = edge_index[1].astype(jnp.int32)
    pad = E_PAD - E
    combo = src * 16384 + dst
    if pad >= 0:
        combo_p = jnp.concatenate([combo, jnp.full((pad,), N, jnp.int32)])
    else:
        combo_p = combo[:E_PAD]
    combo2d = combo_p.reshape(NW * K, C)
    zrow = jnp.zeros((NACC, D), jnp.float32)
    o128 = jnp.ones((C, D), jnp.float32)

    deg8 = _deg(combo2d, zrow, o128)
    agg0 = _agg(x, combo2d, zrow)
    h = _sage_dense(agg0, deg8, x, Wl0, bl0.reshape(1, D), Wr0)
    agg1 = _agg(h, combo2d, zrow)

    Wcp = jnp.pad(Wc, ((0, 0), (0, D - 2)))
    bcp = jnp.pad(bc, (0, D - 2)).reshape(1, D)
    Wa1p = jnp.pad(Wa1, ((0, 0), (0, D - 64)))
    ba1p = jnp.pad(ba1, (0, D - 64)).reshape(1, D)
    Wa2p = jnp.pad(Wa2, ((0, D - 64), (0, D - 2)))
    ba2p = jnp.pad(ba2, (0, D - 2)).reshape(1, D)

    pred_pad, adv_pad = _heads(agg1, deg8, h, Wl1, bl1.reshape(1, D), Wr1,
                               Wcp, bcp, Wa1p, ba1p, Wa2p, ba2p)
    return pred_pad[:, :2], adv_pad[:, :2]


# speedup vs baseline: 3.7054x; 1.2188x over previous
import jax
import jax.numpy as jnp
from jax import lax
from jax.experimental import pallas as pl
from jax.experimental.pallas import tpu as pltpu
from jax.experimental.pallas import tpu_sc as plsc

N = 10000
E = 320000
D = 128
NC, NS = 2, 16
NW = NC * NS
C = 128
K = 80
KF, KS = 128, 32
E_PAD = NW * K * C
CROWS = NS * (KF + KS) + (KF - KS)
NACC = 10112
RPT = NACC // NS
BLK = 1000
GRID = N // BLK


def _sc_mesh():
    return plsc.VectorSubcoreMesh(
        core_axis_name="c", subcore_axis_name="s",
        num_cores=NC, num_subcores=NS)


def _decode_chunk(combo_v, j, src_c, dst_c, b):
    for k in range(C // 16):
        cv = combo_v[j, pl.ds(k * 16, 16)]
        if src_c is not None:
            src_c[b, pl.ds(k * 16, 16)] = lax.shift_right_logical(
                cv, jnp.int32(14))
        dst_c[b, pl.ds(k * 16, 16)] = lax.bitwise_and(cv, jnp.int32(16383))


def _make_sc_aggregate():
    scratch = [
        pltpu.VMEM((KF, C), jnp.int32),
        pltpu.VMEM((2, C), jnp.int32),
        pltpu.VMEM((2, C), jnp.int32),
        pltpu.VMEM((2, C, D), jnp.float32),
        pltpu.VMEM_SHARED((NACC, D), jnp.float32),
        pltpu.SemaphoreType.DMA,
        pltpu.SemaphoreType.DMA,
    ]

    def body(table, combo2d, zrow, agg_out,
             combo_v, src_c, dst_c, rows, acc, sem0, sem1):
        sems = (sem0, sem1)
        cid = lax.axis_index("c")
        sid = lax.axis_index("s")
        r0 = sid * RPT
        kc = jnp.where(cid == 0, KF, KS)
        base = jnp.where(cid == 0, sid * KF, NS * KF + sid * KS)

        pltpu.sync_copy(zrow.at[pl.ds(r0, RPT)], acc.at[pl.ds(r0, RPT)])
        pltpu.sync_copy(combo2d.at[pl.ds(base, KF)], combo_v)
        plsc.subcore_barrier()

        def launch(j, b):
            _decode_chunk(combo_v, j, src_c, dst_c, b)
            pltpu.async_copy(table.at[src_c.at[b]], rows.at[b], sems[b])

        def drain(b):
            pltpu.make_async_copy(
                table.at[src_c.at[b]], rows.at[b], sems[b]).wait()
            pltpu.sync_copy(rows.at[b], acc.at[dst_c.at[b]], add=True)

        launch(0, 0)

        def step(i, carry):
            j = i * 2
            launch(j + 1, 1)
            drain(0)

            @pl.when(j + 2 < kc)
            def _():
                launch(j + 2, 0)

            drain(1)
            return carry

        lax.fori_loop(0, kc // 2, step, 0)
        plsc.subcore_barrier()

        pltpu.sync_copy(acc.at[pl.ds(r0, RPT)],
                        agg_out.at[cid, pl.ds(r0, RPT)])

    return pl.kernel(
        body,
        out_type=jax.ShapeDtypeStruct((NC, NACC, D), jnp.float32),
        mesh=_sc_mesh(),
        scratch_types=scratch,
    )


def _make_sc_degree():
    scratch = [
        pltpu.VMEM((K, C), jnp.int32),
        pltpu.VMEM((1, C), jnp.int32),
        pltpu.VMEM((C, D), jnp.float32),
        pltpu.VMEM_SHARED((NACC, D), jnp.float32),
    ]

    def body(combo2d, zrow, o128, deg_out, combo_v, dst_c, ones_v, degt):
        cid = lax.axis_index("c")
        sid = lax.axis_index("s")
        wid = sid * NC + cid
        r0 = sid * RPT

        pltpu.sync_copy(zrow.at[pl.ds(r0, RPT)], degt.at[pl.ds(r0, RPT)])
        pltpu.sync_copy(combo2d.at[pl.ds(wid * K, K)], combo_v)
        pltpu.sync_copy(o128, ones_v)
        plsc.subcore_barrier()

        def step(j, carry):
            _decode_chunk(combo_v, j, None, dst_c, 0)
            pltpu.sync_copy(ones_v, degt.at[dst_c.at[0]], add=True)
            return carry

        lax.fori_loop(0, K, step, 0)
        plsc.subcore_barrier()
        pltpu.sync_copy(degt.at[pl.ds(r0, RPT)],
                        deg_out.at[cid, pl.ds(r0, RPT)])

    return pl.kernel(
        body,
        out_type=jax.ShapeDtypeStruct((NC, NACC, D), jnp.float32),
        mesh=_sc_mesh(),
        scratch_types=scratch,
    )


def _deg_from(deg_ref):
    return jnp.sum(deg_ref[0] + deg_ref[1], axis=1, keepdims=True) * (1.0 / D)


def _dot(a, b):
    return jnp.dot(a, b, preferred_element_type=jnp.float32,
                   precision=lax.Precision.HIGHEST)


def _sage_dense(agg2, deg8, x, Wl, bl, Wr):
    def body(agg_ref, deg_ref, x_ref, wl_ref, bl_ref, wr_ref, out_ref):
        acc = agg_ref[0] + agg_ref[1]
        deg = jnp.maximum(_deg_from(deg_ref), 1.0)
        mean = acc / deg
        h = _dot(mean, wl_ref[...]) + bl_ref[...] + _dot(x_ref[...], wr_ref[...])
        out_ref[...] = jnp.maximum(h, 0.0)

    return pl.pallas_call(
        body,
        grid=(GRID,),
        in_specs=[
            pl.BlockSpec((NC, BLK, D), lambda i: (0, i, 0)),
            pl.BlockSpec((NC, BLK, D), lambda i: (0, i, 0)),
            pl.BlockSpec((BLK, D), lambda i: (i, 0)),
            pl.BlockSpec((D, D), lambda i: (0, 0)),
            pl.BlockSpec((1, D), lambda i: (0, 0)),
            pl.BlockSpec((D, D), lambda i: (0, 0)),
        ],
        out_specs=pl.BlockSpec((BLK, D), lambda i: (i, 0)),
        out_shape=jax.ShapeDtypeStruct((N, D), jnp.float32),
    )(agg2, deg8, x, Wl, bl, Wr)


def _heads(agg2, deg8, h, Wl, bl, Wr, Wcp, bcp, Wa1p, ba1p, Wa2p, ba2p):
    def body(agg_ref, deg_ref, h_ref, wl_ref, bl_ref, wr_ref,
             wc_ref, bc_ref, wa1_ref, ba1_ref, wa2_ref, ba2_ref,
             pred_ref, adv_ref):
        acc = agg_ref[0] + agg_ref[1]
        deg = jnp.maximum(_deg_from(deg_ref), 1.0)
        mean = acc / deg
        h2 = _dot(mean, wl_ref[...]) + bl_ref[...] + _dot(h_ref[...], wr_ref[...])
        h2 = jnp.maximum(h2, 0.0)
        pred_ref[...] = _dot(h2, wc_ref[...]) + bc_ref[...]
        z = jnp.maximum(_dot(h2, wa1_ref[...]) + ba1_ref[...], 0.0)
        adv_ref[...] = _dot(z, wa2_ref[...]) + ba2_ref[...]

    full = lambda i: (0, 0)
    return pl.pallas_call(
        body,
        grid=(GRID,),
        in_specs=[
            pl.BlockSpec((NC, BLK, D), lambda i: (0, i, 0)),
            pl.BlockSpec((NC, BLK, D), lambda i: (0, i, 0)),
            pl.BlockSpec((BLK, D), lambda i: (i, 0)),
            pl.BlockSpec((D, D), full),
            pl.BlockSpec((1, D), full),
            pl.BlockSpec((D, D), full),
            pl.BlockSpec((D, D), full),
            pl.BlockSpec((1, D), full),
            pl.BlockSpec((D, D), full),
            pl.BlockSpec((1, D), full),
            pl.BlockSpec((D, D), full),
            pl.BlockSpec((1, D), full),
        ],
        out_specs=[
            pl.BlockSpec((BLK, D), lambda i: (i, 0)),
            pl.BlockSpec((BLK, D), lambda i: (i, 0)),
        ],
        out_shape=[
            jax.ShapeDtypeStruct((N, D), jnp.float32),
            jax.ShapeDtypeStruct((N, D), jnp.float32),
        ],
    )(agg2, deg8, h, Wl, bl, Wr, Wcp, bcp, Wa1p, ba1p, Wa2p, ba2p)


_agg = _make_sc_aggregate()
_deg = _make_sc_degree()


def kernel(x, edge_index, Wl0, bl0, Wr0, Wl1, bl1, Wr1, Wc, bc, Wa1, ba1,
           Wa2, ba2):
    src = edge_index[0].astype(jnp.int32)
    dst = edge_index[1].astype(jnp.int32)
    combo = src * 16384 + dst
    pad = CROWS * C - E
    combo2d = jnp.concatenate(
        [combo, jnp.full((pad,), N, jnp.int32)]).reshape(CROWS, C)
    zrow = jnp.zeros((NACC, D), jnp.float32)
    o128 = jnp.ones((C, D), jnp.float32)

    deg8 = _deg(combo2d, zrow, o128)
    agg0 = _agg(x, combo2d, zrow)
    h = _sage_dense(agg0, deg8, x, Wl0, bl0.reshape(1, D), Wr0)
    agg1 = _agg(h, combo2d, zrow)

    Wcp = jnp.pad(Wc, ((0, 0), (0, D - 2)))
    bcp = jnp.pad(bc, (0, D - 2)).reshape(1, D)
    Wa1p = jnp.pad(Wa1, ((0, 0), (0, D - 64)))
    ba1p = jnp.pad(ba1, (0, D - 64)).reshape(1, D)
    Wa2p = jnp.pad(Wa2, ((0, D - 64), (0, D - 2)))
    ba2p = jnp.pad(ba2, (0, D - 2)).reshape(1, D)

    pred_pad, adv_pad = _heads(agg1, deg8, h, Wl1, bl1.reshape(1, D), Wr1,
                               Wcp, bcp, Wa1p, ba1p, Wa2p, ba2p)
    return pred_pad[:, :2], adv_pad[:, :2]
